# baseline (speedup 1.0000x reference)
import jax
import jax.numpy as jnp
from jax import lax
from jax.experimental import pallas as pl
from jax.experimental.pallas import tpu as pltpu
from jax.experimental.pallas import tpu_sc as plsc

NUM_TERMS = 16
NUM_COLS = 16384
DEPTH = 128
NUM_WORKERS = 32
COLS_PER_W = NUM_COLS // NUM_WORKERS
CHUNK = 128
NUM_CHUNKS = COLS_PER_W // CHUNK
LANES = 16
VECS_PER_ROW = DEPTH // LANES


def _body(x_hbm, pe_hbm, out_hbm,
          idx_v, acc0, acc1, acc2, acc3, stage0, stage1,
          gsem0, gsem1, gsem2, gsem3, wsem0, wsem1):
    cid = lax.axis_index("c")
    sid = lax.axis_index("s")
    wid = sid * 2 + cid
    col0 = wid * COLS_PER_W
    inv = jnp.float32(1.0 / NUM_TERMS)
    zvec = jnp.zeros((LANES,), jnp.float32)

    accs = (acc0, acc1, acc2, acc3)
    stages = (stage0, stage1)
    gsems = (gsem0, gsem1, gsem2, gsem3)
    wsems = (wsem0, wsem1)

    def gather(k, i, add):
        return pltpu.async_copy(
            pe_hbm.at[idx_v.at[i, pl.ds(k * CHUNK, CHUNK)]],
            accs[k], gsems[k], add=add)

    pltpu.sync_copy(x_hbm.at[:, pl.ds(col0, CHUNK)], idx_v.at[:, pl.ds(0, CHUNK)])
    first = {0: gather(0, 0, False)}
    pltpu.sync_copy(x_hbm.at[:, pl.ds(col0 + CHUNK, COLS_PER_W - CHUNK)],
                    idx_v.at[:, pl.ds(CHUNK, COLS_PER_W - CHUNK)])
    for k in range(1, NUM_CHUNKS):
        first[k] = gather(k, 0, False)
    pending = {}
    for k in range(NUM_CHUNKS):
        first.pop(k).wait()
        pending[k] = [gather(k, i, True) for i in range(1, NUM_TERMS)]

    wb = {}
    for k in range(NUM_CHUNKS):
        acc, stage = accs[k], stages[k % 2]
        for cd in pending.pop(k):
            cd.wait()
        if k - 2 in wb:
            wb.pop(k - 2).wait()

        def row_body(r2, carry):
            for r in (2 * r2, 2 * r2 + 1):
                for j in range(VECS_PER_ROW):
                    sl = pl.ds(j * LANES, LANES)
                    stage[r, sl] = acc[r, sl] * inv
            return carry

        lax.fori_loop(0, CHUNK // 2, row_body, 0)
        wb[k] = pltpu.async_copy(
            stage, out_hbm.at[pl.ds(col0 + k * CHUNK, CHUNK)], wsems[k % 2])
    for k in sorted(wb):
        wb.pop(k).wait()


@jax.jit
def kernel(x, position_encoding):
    mesh = plsc.VectorSubcoreMesh(core_axis_name="c", subcore_axis_name="s")
    f = pl.kernel(
        _body,
        mesh=mesh,
        out_type=jax.ShapeDtypeStruct((NUM_COLS, DEPTH), jnp.float32),
        scratch_types=[
            pltpu.VMEM((NUM_TERMS, COLS_PER_W), jnp.int32),
            pltpu.VMEM((CHUNK, DEPTH), jnp.float32),
            pltpu.VMEM((CHUNK, DEPTH), jnp.float32),
            pltpu.VMEM((CHUNK, DEPTH), jnp.float32),
            pltpu.VMEM((CHUNK, DEPTH), jnp.float32),
            pltpu.VMEM((CHUNK, DEPTH), jnp.float32),
            pltpu.VMEM((CHUNK, DEPTH), jnp.float32),
            pltpu.SemaphoreType.DMA,
            pltpu.SemaphoreType.DMA,
            pltpu.SemaphoreType.DMA,
            pltpu.SemaphoreType.DMA,
            pltpu.SemaphoreType.DMA,
            pltpu.SemaphoreType.DMA,
        ],
    )
    return f(x, position_encoding)
